# Initial kernel scaffold; baseline (speedup 1.0000x reference)
import jax
import jax.numpy as jnp
from jax.experimental import pallas as pl


def kernel(x, seg, emot, training, word_table, pos_table, seg_table, emot_table, gamma, beta):
    raise NotImplementedError("write your pallas kernel here")


# speedup vs baseline: 3.4917x; 3.4917x over previous
import functools
import jax
import jax.numpy as jnp
from jax import lax
from jax.experimental import pallas as pl
from jax.experimental.pallas import tpu as pltpu
from jax.experimental.pallas import tpu_sc as plsc

B, L, D = 1024, 200, 64
VOCAB = 100000
PADDING_IDX = 1
LN_EPS = 1e-6

NC, NS = 2, 16
NW = NC * NS
TOKENS = B * L
TPW = TOKENS // NW
CHUNK = 128
NCHUNK = TPW // CHUNK


def _rsqrt(x):
    i = lax.bitcast_convert_type(x, jnp.int32)
    i = jnp.int32(0x5F3759DF) - lax.shift_right_arithmetic(i, 1)
    y = lax.bitcast_convert_type(i, jnp.float32)
    for _ in range(3):
        y = y * (1.5 - 0.5 * x * y * y)
    return y


def _allsum(v, perms):
    for p in perms:
        v = v + jnp.take_along_axis(v, p, axis=0)
    return v


def _wid():
    return lax.axis_index("s") * NC + lax.axis_index("c")


def _body(x_hbm, seg_hbm, emot_hbm, word_hbm, pos_hbm, segt_hbm, emott_hbm,
          gamma_hbm, beta_hbm, out_hbm,
          xb, sb, eb, rows, pseg, emott, posb, segtb, gb, bb, gsem):
    wid = _wid()
    iota16 = lax.iota(jnp.int32, 16)
    perms = [lax.bitwise_xor(iota16, jnp.int32(k)) for k in (8, 4, 2, 1)]

    pltpu.sync_copy(pos_hbm, posb)
    pltpu.sync_copy(segt_hbm, segtb)
    pltpu.sync_copy(emott_hbm, emott)
    pltpu.sync_copy(gamma_hbm, gb)
    pltpu.sync_copy(beta_hbm, bb)

    gamma_v = [gb[pl.ds(c * 16, 16)] for c in range(4)]
    beta_v = [bb[pl.ds(c * 16, 16)] for c in range(4)]
    seg0 = [segtb[0, pl.ds(c * 16, 16)] for c in range(4)]
    seg1 = [segtb[1, pl.ds(c * 16, 16)] for c in range(4)]

    def build(l, _):
        for c in range(4):
            p = posb[l, pl.ds(c * 16, 16)]
            pseg[2 * l, pl.ds(c * 16, 16)] = p + seg0[c]
            pseg[2 * l + 1, pl.ds(c * 16, 16)] = p + seg1[c]
        return _

    lax.fori_loop(0, L, build, None)

    def chunk_body(ci, _):
        base = wid * TPW + ci * CHUNK
        pltpu.sync_copy(x_hbm.at[pl.ds(base, CHUNK)], xb)
        pltpu.sync_copy(seg_hbm.at[pl.ds(base, CHUNK)], sb)
        pltpu.sync_copy(emot_hbm.at[pl.ds(base, CHUNK)], eb)
        pltpu.async_copy(word_hbm.at[xb], rows, gsem).wait()
        l0 = lax.rem(ci * CHUNK, L)

        def group(g, _):
            tbase = g * 16
            sv = sb[pl.ds(tbase, 16)]
            ev = eb[pl.ds(tbase, 16)]
            for j in range(16):
                t = tbase + j
                l = lax.rem(l0 + t, L)
                ps_row = 2 * l + sv[j]
                e_row = ev[j]
                h = [rows[t, pl.ds(c * 16, 16)]
                     + pseg[ps_row, pl.ds(c * 16, 16)]
                     + emott[e_row, pl.ds(c * 16, 16)] for c in range(4)]
                ssum = _allsum(h[0] + h[1] + h[2] + h[3], perms)
                qsum = _allsum(h[0] * h[0] + h[1] * h[1]
                               + h[2] * h[2] + h[3] * h[3], perms)
                mean_b = ssum * (1.0 / D)
                var_b = qsum * (1.0 / D) - mean_b * mean_b
                rstd_b = _rsqrt(var_b + LN_EPS)
                for c in range(4):
                    y = (h[c] - mean_b) * rstd_b * gamma_v[c] + beta_v[c]
                    rows[t, pl.ds(c * 16, 16)] = y
            return _

        lax.fori_loop(0, CHUNK // 16, group, None)
        pltpu.sync_copy(rows, out_hbm.at[pl.ds(base, CHUNK)])
        return _

    lax.fori_loop(0, NCHUNK, chunk_body, None)


@jax.jit
def _embed_ln(xf, sf, ef, word_table, pos_slice, seg_table, emot_table,
              gamma, beta):
    mesh = plsc.VectorSubcoreMesh(core_axis_name="c", subcore_axis_name="s",
                                  num_cores=NC, num_subcores=NS)
    return pl.kernel(
        _body,
        out_type=jax.ShapeDtypeStruct((TOKENS, D), jnp.float32),
        mesh=mesh,
        compiler_params=pltpu.CompilerParams(use_tc_tiling_on_sc=False),
        scratch_types=[
            pltpu.VMEM((CHUNK,), jnp.int32),
            pltpu.VMEM((CHUNK,), jnp.int32),
            pltpu.VMEM((CHUNK,), jnp.int32),
            pltpu.VMEM((CHUNK, D), jnp.float32),
            pltpu.VMEM((2 * L, D), jnp.float32),
            pltpu.VMEM((41, D), jnp.float32),
            pltpu.VMEM((L, D), jnp.float32),
            pltpu.VMEM((2, D), jnp.float32),
            pltpu.VMEM((D,), jnp.float32),
            pltpu.VMEM((D,), jnp.float32),
            pltpu.SemaphoreType.DMA,
        ],
    )(xf, sf, ef, word_table, pos_slice, seg_table, emot_table, gamma, beta)


def kernel(x, seg, emot, training, word_table, pos_table, seg_table,
           emot_table, gamma, beta):
    xf = x.reshape(-1).astype(jnp.int32)
    sf = seg.reshape(-1).astype(jnp.int32)
    ef = emot.reshape(-1).astype(jnp.int32)
    pos_slice = lax.slice(pos_table, (PADDING_IDX + 1, 0),
                          (L + PADDING_IDX + 1, D))
    out = _embed_ln(xf, sf, ef, word_table, pos_slice, seg_table,
                    emot_table, gamma, beta)
    return out.reshape(B, L, D)


# speedup vs baseline: 4.1823x; 1.1978x over previous
import functools
import jax
import jax.numpy as jnp
from jax import lax
from jax.experimental import pallas as pl
from jax.experimental.pallas import tpu as pltpu
from jax.experimental.pallas import tpu_sc as plsc

B, L, D = 1024, 200, 64
VOCAB = 100000
PADDING_IDX = 1
LN_EPS = 1e-6

NC, NS = 2, 16
NW = NC * NS
TOKENS = B * L
TPW = TOKENS // NW
CHUNK = 128
NCHUNK = TPW // CHUNK
NPAIR = NCHUNK // 2


def _rsqrt(x):
    i = lax.bitcast_convert_type(x, jnp.int32)
    i = jnp.int32(0x5F3759DF) - lax.shift_right_arithmetic(i, 1)
    y = lax.bitcast_convert_type(i, jnp.float32)
    for _ in range(3):
        y = y * (1.5 - 0.5 * x * y * y)
    return y


def _allsum(v, perms):
    for p in perms:
        v = v + jnp.take_along_axis(v, p, axis=0)
    return v


def _wid():
    return lax.axis_index("s") * NC + lax.axis_index("c")


def _body(x_hbm, seg_hbm, emot_hbm, word_hbm, pos_hbm, segt_hbm, emott_hbm,
          gamma_hbm, beta_hbm, out_hbm,
          xb, sbm, ebm, rowsA, rowsB, obufA, obufB, pseg, emott, posb, segtb,
          gb, bb, gsemA, gsemB, osemA, osemB):
    wid = _wid()
    iota16 = lax.iota(jnp.int32, 16)
    perms = [lax.bitwise_xor(iota16, jnp.int32(k)) for k in (8, 4, 2, 1)]
    base0 = wid * TPW

    pltpu.sync_copy(x_hbm.at[wid], xb)
    pltpu.sync_copy(seg_hbm.at[wid], sbm)
    pltpu.sync_copy(emot_hbm.at[wid], ebm)
    pltpu.sync_copy(pos_hbm, posb)
    pltpu.sync_copy(segt_hbm, segtb)
    pltpu.sync_copy(emott_hbm, emott)
    pltpu.sync_copy(gamma_hbm, gb)
    pltpu.sync_copy(beta_hbm, bb)

    gamma_v = [gb[pl.ds(c * 16, 16)] for c in range(4)]
    beta_v = [bb[pl.ds(c * 16, 16)] for c in range(4)]
    seg0 = [segtb[0, pl.ds(c * 16, 16)] for c in range(4)]
    seg1 = [segtb[1, pl.ds(c * 16, 16)] for c in range(4)]

    def build(l, _):
        for c in range(4):
            p = posb[l, pl.ds(c * 16, 16)]
            pseg[2 * l, pl.ds(c * 16, 16)] = p + seg0[c]
            pseg[2 * l + 1, pl.ds(c * 16, 16)] = p + seg1[c]
        return _

    lax.fori_loop(0, L, build, None)

    def gather(c, rows, sem):
        return pltpu.make_async_copy(word_hbm.at[xb.at[c]], rows, sem)

    def scatter(c, obuf, sem):
        dst = out_hbm.at[pl.ds(base0 + c * CHUNK, CHUNK)]
        return pltpu.make_async_copy(obuf, dst, sem)

    def compute(ci, rows, obuf):
        l0 = lax.rem(ci * CHUNK, L)

        def group(g, _):
            tbase = g * 16
            sv = sbm[ci, pl.ds(tbase, 16)]
            ev = ebm[ci, pl.ds(tbase, 16)]
            for j in range(16):
                t = tbase + j
                l = lax.rem(l0 + t, L)
                ps_row = 2 * l + sv[j]
                e_row = ev[j]
                h = [rows[t, pl.ds(c * 16, 16)]
                     + pseg[ps_row, pl.ds(c * 16, 16)]
                     + emott[e_row, pl.ds(c * 16, 16)] for c in range(4)]
                ssum = _allsum(h[0] + h[1] + h[2] + h[3], perms)
                qsum = _allsum(h[0] * h[0] + h[1] * h[1]
                               + h[2] * h[2] + h[3] * h[3], perms)
                mean_b = ssum * (1.0 / D)
                var_b = qsum * (1.0 / D) - mean_b * mean_b
                rstd_b = _rsqrt(var_b + LN_EPS)
                for c in range(4):
                    y = (h[c] - mean_b) * rstd_b * gamma_v[c] + beta_v[c]
                    obuf[t, pl.ds(c * 16, 16)] = y
            return _

        lax.fori_loop(0, CHUNK // 16, group, None)

    gather(0, rowsA, gsemA).start()

    def pair(c2, _):
        a = 2 * c2
        b = a + 1
        gather(b, rowsB, gsemB).start()
        gather(a, rowsA, gsemA).wait()

        @pl.when(c2 > 0)
        def _w1():
            scatter(a - 2, obufA, osemA).wait()

        compute(a, rowsA, obufA)
        scatter(a, obufA, osemA).start()

        @pl.when(c2 < NPAIR - 1)
        def _g1():
            gather(a + 2, rowsA, gsemA).start()

        gather(b, rowsB, gsemB).wait()

        @pl.when(c2 > 0)
        def _w2():
            scatter(b - 2, obufB, osemB).wait()

        compute(b, rowsB, obufB)
        scatter(b, obufB, osemB).start()
        return _

    lax.fori_loop(0, NPAIR, pair, None)
    scatter(NCHUNK - 2, obufA, osemA).wait()
    scatter(NCHUNK - 1, obufB, osemB).wait()


@jax.jit
def _embed_ln(xf, sf, ef, word_table, pos_slice, seg_table, emot_table,
              gamma, beta):
    mesh = plsc.VectorSubcoreMesh(core_axis_name="c", subcore_axis_name="s",
                                  num_cores=NC, num_subcores=NS)
    return pl.kernel(
        _body,
        out_type=jax.ShapeDtypeStruct((TOKENS, D), jnp.float32),
        mesh=mesh,
        compiler_params=pltpu.CompilerParams(use_tc_tiling_on_sc=False),
        scratch_types=[
            pltpu.VMEM((NCHUNK, CHUNK), jnp.int32),
            pltpu.VMEM((NCHUNK, CHUNK), jnp.int32),
            pltpu.VMEM((NCHUNK, CHUNK), jnp.int32),
            pltpu.VMEM((CHUNK, D), jnp.float32),
            pltpu.VMEM((CHUNK, D), jnp.float32),
            pltpu.VMEM((CHUNK, D), jnp.float32),
            pltpu.VMEM((CHUNK, D), jnp.float32),
            pltpu.VMEM((2 * L, D), jnp.float32),
            pltpu.VMEM((41, D), jnp.float32),
            pltpu.VMEM((L, D), jnp.float32),
            pltpu.VMEM((2, D), jnp.float32),
            pltpu.VMEM((D,), jnp.float32),
            pltpu.VMEM((D,), jnp.float32),
            pltpu.SemaphoreType.DMA,
            pltpu.SemaphoreType.DMA,
            pltpu.SemaphoreType.DMA,
            pltpu.SemaphoreType.DMA,
        ],
    )(xf, sf, ef, word_table, pos_slice, seg_table, emot_table, gamma, beta)


def kernel(x, seg, emot, training, word_table, pos_table, seg_table,
           emot_table, gamma, beta):
    xf = x.reshape(NW, NCHUNK, CHUNK).astype(jnp.int32)
    sf = seg.reshape(NW, NCHUNK, CHUNK).astype(jnp.int32)
    ef = emot.reshape(NW, NCHUNK, CHUNK).astype(jnp.int32)
    pos_slice = lax.slice(pos_table, (PADDING_IDX + 1, 0),
                          (L + PADDING_IDX + 1, D))
    out = _embed_ln(xf, sf, ef, word_table, pos_slice, seg_table,
                    emot_table, gamma, beta)
    return out.reshape(B, L, D)


# speedup vs baseline: 6.8697x; 1.6426x over previous
import functools
import jax
import jax.numpy as jnp
from jax import lax
from jax.experimental import pallas as pl
from jax.experimental.pallas import tpu as pltpu
from jax.experimental.pallas import tpu_sc as plsc

B, L, D = 1024, 200, 64
VOCAB = 100000
PADDING_IDX = 1
LN_EPS = 1e-6

NC, NS = 2, 16
NW = NC * NS
TOKENS = B * L
TPW = TOKENS // NW
CHUNK = 128
NCHUNK = TPW // CHUNK
NPAIR = NCHUNK // 2
PSEG_L = L - 8 + CHUNK


def _rsqrt(x):
    i = lax.bitcast_convert_type(x, jnp.int32)
    i = jnp.int32(0x5F3759DF) - lax.shift_right_arithmetic(i, 1)
    y = lax.bitcast_convert_type(i, jnp.float32)
    for _ in range(2):
        y = y * (1.5 - 0.5 * x * y * y)
    return y


def _allsum(v, perms):
    for p in perms:
        v = v + jnp.take_along_axis(v, p, axis=0)
    return v


def _wid():
    return lax.axis_index("s") * NC + lax.axis_index("c")


def _body(x_hbm, seg_hbm, emot_hbm, word_hbm, pos_hbm, segt_hbm, emott_hbm,
          gamma_hbm, beta_hbm, out_hbm,
          xb, sbm, ebm, rowsA, rowsB, obufA, obufB, pseg, emott, posb, segtb,
          gb, bb, gsemA, gsemB, osemA, osemB):
    wid = _wid()
    iota16 = lax.iota(jnp.int32, 16)
    perms = [lax.bitwise_xor(iota16, jnp.int32(k)) for k in (8, 4, 2, 1)]
    base0 = wid * TPW

    pltpu.sync_copy(x_hbm.at[wid], xb)
    pltpu.sync_copy(seg_hbm.at[wid], sbm)
    pltpu.sync_copy(emot_hbm.at[wid], ebm)
    pltpu.sync_copy(pos_hbm, posb)
    pltpu.sync_copy(segt_hbm, segtb)
    pltpu.sync_copy(emott_hbm, emott)
    pltpu.sync_copy(gamma_hbm, gb)
    pltpu.sync_copy(beta_hbm, bb)

    gamma_v = [gb[pl.ds(c * 16, 16)] for c in range(4)]
    beta_v = [bb[pl.ds(c * 16, 16)] for c in range(4)]
    seg0 = [segtb[0, pl.ds(c * 16, 16)] for c in range(4)]
    seg1 = [segtb[1, pl.ds(c * 16, 16)] for c in range(4)]

    def build(l, _):
        lsrc = lax.rem(l, L)
        for c in range(4):
            p = posb[lsrc, pl.ds(c * 16, 16)]
            pseg[2 * l, pl.ds(c * 16, 16)] = p + seg0[c]
            pseg[2 * l + 1, pl.ds(c * 16, 16)] = p + seg1[c]
        return _

    lax.fori_loop(0, PSEG_L, build, None)

    def gather(c, rows, sem):
        return pltpu.make_async_copy(word_hbm.at[xb.at[c]], rows, sem)

    def scatter(c, obuf, sem):
        dst = out_hbm.at[pl.ds(base0 + c * CHUNK, CHUNK)]
        return pltpu.make_async_copy(obuf, dst, sem)

    def compute(ci, rows, obuf):
        l0 = lax.rem(ci * CHUNK, L)
        zero16 = iota16 * 0

        def group(g, _):
            tbase = g * 16
            rb = 2 * (l0 + tbase)
            sv = sbm[ci, pl.ds(tbase, 16)]
            ev = ebm[ci, pl.ds(tbase, 16)]
            for half in range(2):
                sumv = jnp.full((16,), 0.0, jnp.float32)
                sqv = jnp.full((16,), 0.0, jnp.float32)
                hs = []
                for jj in range(8):
                    j = half * 8 + jj
                    t = tbase + j
                    ps_row = rb + 2 * j + sv[j]
                    e_row = ev[j]
                    h = [rows[t, pl.ds(c * 16, 16)]
                         + pseg[ps_row, pl.ds(c * 16, 16)]
                         + emott[e_row, pl.ds(c * 16, 16)] for c in range(4)]
                    ssum = _allsum(h[0] + h[1] + h[2] + h[3], perms)
                    qsum = _allsum(h[0] * h[0] + h[1] * h[1]
                                   + h[2] * h[2] + h[3] * h[3], perms)
                    mj = iota16 == jj
                    sumv = jnp.where(mj, ssum, sumv)
                    sqv = jnp.where(mj, qsum, sqv)
                    hs.append(h)
                meanv = sumv * (1.0 / D)
                varv = sqv * (1.0 / D) - meanv * meanv
                rstdv = _rsqrt(varv + LN_EPS)
                for jj in range(8):
                    t = tbase + half * 8 + jj
                    idxj = zero16 + jj
                    mean_b = jnp.take_along_axis(meanv, idxj, axis=0)
                    rstd_b = jnp.take_along_axis(rstdv, idxj, axis=0)
                    for c in range(4):
                        y = ((hs[jj][c] - mean_b) * rstd_b
                             * gamma_v[c] + beta_v[c])
                        obuf[t, pl.ds(c * 16, 16)] = y
            return _

        lax.fori_loop(0, CHUNK // 16, group, None)

    gather(0, rowsA, gsemA).start()

    def pair(c2, _):
        a = 2 * c2
        b = a + 1
        gather(b, rowsB, gsemB).start()
        gather(a, rowsA, gsemA).wait()

        @pl.when(c2 > 0)
        def _w1():
            scatter(a - 2, obufA, osemA).wait()

        compute(a, rowsA, obufA)
        scatter(a, obufA, osemA).start()

        @pl.when(c2 < NPAIR - 1)
        def _g1():
            gather(a + 2, rowsA, gsemA).start()

        gather(b, rowsB, gsemB).wait()

        @pl.when(c2 > 0)
        def _w2():
            scatter(b - 2, obufB, osemB).wait()

        compute(b, rowsB, obufB)
        scatter(b, obufB, osemB).start()
        return _

    lax.fori_loop(0, NPAIR, pair, None)
    scatter(NCHUNK - 2, obufA, osemA).wait()
    scatter(NCHUNK - 1, obufB, osemB).wait()


@jax.jit
def _embed_ln(xf, sf, ef, word_table, pos_slice, seg_table, emot_table,
              gamma, beta):
    mesh = plsc.VectorSubcoreMesh(core_axis_name="c", subcore_axis_name="s",
                                  num_cores=NC, num_subcores=NS)
    return pl.kernel(
        _body,
        out_type=jax.ShapeDtypeStruct((TOKENS, D), jnp.float32),
        mesh=mesh,
        compiler_params=pltpu.CompilerParams(use_tc_tiling_on_sc=False),
        scratch_types=[
            pltpu.VMEM((NCHUNK, CHUNK), jnp.int32),
            pltpu.VMEM((NCHUNK, CHUNK), jnp.int32),
            pltpu.VMEM((NCHUNK, CHUNK), jnp.int32),
            pltpu.VMEM((CHUNK, D), jnp.float32),
            pltpu.VMEM((CHUNK, D), jnp.float32),
            pltpu.VMEM((CHUNK, D), jnp.float32),
            pltpu.VMEM((CHUNK, D), jnp.float32),
            pltpu.VMEM((2 * PSEG_L, D), jnp.float32),
            pltpu.VMEM((41, D), jnp.float32),
            pltpu.VMEM((L, D), jnp.float32),
            pltpu.VMEM((2, D), jnp.float32),
            pltpu.VMEM((D,), jnp.float32),
            pltpu.VMEM((D,), jnp.float32),
            pltpu.SemaphoreType.DMA,
            pltpu.SemaphoreType.DMA,
            pltpu.SemaphoreType.DMA,
            pltpu.SemaphoreType.DMA,
        ],
    )(xf, sf, ef, word_table, pos_slice, seg_table, emot_table, gamma, beta)


def kernel(x, seg, emot, training, word_table, pos_table, seg_table,
           emot_table, gamma, beta):
    xf = x.reshape(NW, NCHUNK, CHUNK).astype(jnp.int32)
    sf = seg.reshape(NW, NCHUNK, CHUNK).astype(jnp.int32)
    ef = emot.reshape(NW, NCHUNK, CHUNK).astype(jnp.int32)
    pos_slice = lax.slice(pos_table, (PADDING_IDX + 1, 0),
                          (L + PADDING_IDX + 1, D))
    out = _embed_ln(xf, sf, ef, word_table, pos_slice, seg_table,
                    emot_table, gamma, beta)
    return out.reshape(B, L, D)
